# Initial kernel scaffold; baseline (speedup 1.0000x reference)
import jax
import jax.numpy as jnp
from jax.experimental import pallas as pl


def kernel(x, mask_rows, mask_cols, kernel, bias):
    raise NotImplementedError("write your pallas kernel here")


# speedup vs baseline: 3.5083x; 3.5083x over previous
import dataclasses
import functools

import jax
import jax.numpy as jnp
from jax import lax
from jax.experimental import pallas as pl
from jax.experimental.pallas import tpu as pltpu
from jax.experimental.pallas import tpu_sc as plsc

B = 16
IN_LEN = 50000
OUT_LEN = 5000
NNZ = 1600000
NCORES = 2
NSUB = 16
LANES = 16
COL_BOUND = 2560
HALF_LEN = 2560
OUT_PAD = 2 * HALF_LEN
CHUNK = 1024


def _body(x_hbm, rows_hbm, cols_hbm, w_hbm, bias_hbm, off_hbm, out_hbm,
          xb, accv, rbuf, cbuf, wbuf, offv, bbuf):
    c_idx = lax.axis_index("c")
    s_idx = lax.axis_index("s")
    wid = s_idx * NCORES + c_idx
    b = wid % B
    h = wid // B

    pltpu.sync_copy(x_hbm.at[b], xb)
    pltpu.sync_copy(off_hbm, offv)

    iot = lax.iota(jnp.int32, LANES)
    ov = offv[...]
    n_lo = jnp.sum(jnp.where(iot == h, ov, 0))
    n_hi = jnp.sum(jnp.where(iot == h + 1, ov, 0))

    c0 = h * HALF_LEN
    pltpu.sync_copy(bias_hbm.at[pl.ds(c0, HALF_LEN)], bbuf)

    @pl.loop(0, HALF_LEN, step=LANES)
    def _init(j):
        accv[pl.ds(c0 + j, LANES)] = bbuf[pl.ds(j, LANES)]

    lo_al = n_lo & ~7
    nchunks = (n_hi - lo_al + CHUNK - 1) // CHUNK

    def chunk_body(k, carry):
        start = lo_al + k * CHUNK
        base = jnp.minimum(start, NNZ - CHUNK)
        base = pl.multiple_of(base, 8)
        pltpu.sync_copy(rows_hbm.at[pl.ds(base, CHUNK)], rbuf)
        pltpu.sync_copy(cols_hbm.at[pl.ds(base, CHUNK)], cbuf)
        pltpu.sync_copy(w_hbm.at[pl.ds(base, CHUNK)], wbuf)
        lb = jnp.maximum(n_lo, start)

        @pl.loop(0, CHUNK, step=LANES)
        def _inner(j):
            g = (base + j) + iot
            m = (g >= lb) & (g < n_hi)
            r = rbuf[pl.ds(j, LANES)]
            xv = plsc.load_gather(xb, [r])
            wv = wbuf[pl.ds(j, LANES)]
            cv = cbuf[pl.ds(j, LANES)]
            plsc.addupdate_scatter(accv, [cv], xv * wv, mask=m)

        return carry

    lax.fori_loop(0, nchunks, chunk_body, 0)

    pltpu.sync_copy(accv.at[pl.ds(c0, HALF_LEN)],
                    out_hbm.at[b, pl.ds(c0, HALF_LEN)])


@jax.jit
def kernel(x, mask_rows, mask_cols, kernel, bias):
    x2 = x.reshape(B, IN_LEN)
    bias_pad = jnp.pad(bias[:, 0], (0, OUT_PAD - OUT_LEN))
    mid = jnp.searchsorted(mask_cols, COL_BOUND).astype(jnp.int32)
    off = jnp.zeros((LANES,), jnp.int32)
    off = off.at[1].set(mid)
    off = off.at[2:].set(NNZ)

    mesh = plsc.VectorSubcoreMesh(core_axis_name="c", subcore_axis_name="s")
    cp = pltpu.CompilerParams()
    if "needs_layout_passes" in pltpu.CompilerParams.__dataclass_fields__:
        cp = dataclasses.replace(cp, needs_layout_passes=False)
    run = functools.partial(
        pl.kernel,
        compiler_params=cp,
        out_type=jax.ShapeDtypeStruct((B, OUT_PAD), jnp.float32),
        mesh=mesh,
        scratch_types=[
            pltpu.VMEM((IN_LEN,), jnp.float32),
            pltpu.VMEM((OUT_PAD,), jnp.float32),
            pltpu.VMEM((CHUNK,), jnp.int32),
            pltpu.VMEM((CHUNK,), jnp.int32),
            pltpu.VMEM((CHUNK,), jnp.float32),
            pltpu.VMEM((LANES,), jnp.int32),
            pltpu.VMEM((HALF_LEN,), jnp.float32),
        ],
    )(_body)
    outp = run(x2, mask_rows, mask_cols, kernel, bias_pad, off)
    return outp[:, :OUT_LEN].reshape(B, OUT_LEN, 1)


# speedup vs baseline: 7.0781x; 2.0176x over previous
import dataclasses
import functools

import jax
import jax.numpy as jnp
from jax import lax
from jax.experimental import pallas as pl
from jax.experimental.pallas import tpu as pltpu
from jax.experimental.pallas import tpu_sc as plsc

B = 16
IN_LEN = 50000
OUT_LEN = 5000
NNZ = 1600000
NCORES = 2
LANES = 16
HALF_LEN = 2560
OUT_PAD = 2 * HALF_LEN
CHUNK = 2048
UNROLL = 4


def _body(x_hbm, rows_hbm, cols_hbm, w_hbm, bias_hbm, off_hbm, out_hbm,
          xb, accv, rbuf0, cbuf0, wbuf0, rbuf1, cbuf1, wbuf1, offv, bbuf,
          sem_a, sem_b):
    c_idx = lax.axis_index("c")
    s_idx = lax.axis_index("s")
    wid = s_idx * NCORES + c_idx
    b = wid % B
    h = wid // B

    pltpu.sync_copy(x_hbm.at[b], xb)
    pltpu.sync_copy(off_hbm, offv)

    iot = lax.iota(jnp.int32, LANES)
    ov = offv[...]
    n_lo = jnp.sum(jnp.where(iot == h, ov, 0))
    n_hi = jnp.sum(jnp.where(iot == h + 1, ov, 0))

    c0 = h * HALF_LEN
    pltpu.sync_copy(bias_hbm.at[pl.ds(c0, HALF_LEN)], bbuf)

    @pl.loop(0, HALF_LEN, step=LANES)
    def _init(j):
        accv[pl.ds(c0 + j, LANES)] = bbuf[pl.ds(j, LANES)]

    lo_al = n_lo & ~7
    nchunks = (n_hi - lo_al + CHUNK - 1) // CHUNK

    def chunk_base(k):
        start = lo_al + k * CHUNK
        base = jnp.minimum(start, NNZ - CHUNK)
        return pl.multiple_of(base, 8)

    def copies(k, rb, cb, wb, sem):
        base = chunk_base(k)
        return (
            pltpu.make_async_copy(rows_hbm.at[pl.ds(base, CHUNK)], rb, sem),
            pltpu.make_async_copy(cols_hbm.at[pl.ds(base, CHUNK)], cb, sem),
            pltpu.make_async_copy(w_hbm.at[pl.ds(base, CHUNK)], wb, sem),
        )

    def issue(k, rb, cb, wb, sem):
        for c in copies(k, rb, cb, wb, sem):
            c.start()

    def drain(k, rb, cb, wb, sem):
        for c in copies(k, rb, cb, wb, sem):
            c.wait()

    def compute(k, rb, cb, wb):
        start = lo_al + k * CHUNK
        base = chunk_base(k)
        interior = (start >= n_lo) & (start + CHUNK <= n_hi)

        @pl.when(interior)
        def _fast():
            @pl.loop(0, CHUNK, step=UNROLL * LANES)
            def _grp(j):
                for u in range(UNROLL):
                    sl = pl.ds(j + u * LANES, LANES)
                    xv = plsc.load_gather(xb, [rb[sl]])
                    plsc.addupdate_scatter(accv, [cb[sl]], xv * wb[sl])

        @pl.when(jnp.logical_not(interior))
        def _masked():
            lb = jnp.maximum(n_lo, start)

            @pl.loop(0, CHUNK, step=LANES)
            def _grp(j):
                g = (base + j) + iot
                m = (g >= lb) & (g < n_hi)
                sl = pl.ds(j, LANES)
                xv = plsc.load_gather(xb, [rb[sl]])
                plsc.addupdate_scatter(accv, [cb[sl]], xv * wb[sl], mask=m)

    issue(0, rbuf0, cbuf0, wbuf0, sem_a)
    npairs = (nchunks + 1) // 2

    def pair(p, carry):
        k0 = 2 * p
        drain(k0, rbuf0, cbuf0, wbuf0, sem_a)
        issue(k0 + 1, rbuf1, cbuf1, wbuf1, sem_b)
        compute(k0, rbuf0, cbuf0, wbuf0)
        drain(k0 + 1, rbuf1, cbuf1, wbuf1, sem_b)
        issue(k0 + 2, rbuf0, cbuf0, wbuf0, sem_a)
        compute(k0 + 1, rbuf1, cbuf1, wbuf1)
        return carry

    lax.fori_loop(0, npairs, pair, 0)
    drain(2 * npairs, rbuf0, cbuf0, wbuf0, sem_a)

    pltpu.sync_copy(accv.at[pl.ds(c0, HALF_LEN)],
                    out_hbm.at[b, pl.ds(c0, HALF_LEN)])


@jax.jit
def kernel(x, mask_rows, mask_cols, kernel, bias):
    x2 = x.reshape(B, IN_LEN)
    bias_pad = jnp.pad(bias[:, 0], (0, OUT_PAD - OUT_LEN))
    mid = jnp.searchsorted(mask_cols, HALF_LEN).astype(jnp.int32)
    off = jnp.zeros((LANES,), jnp.int32)
    off = off.at[1].set(mid)
    off = off.at[2:].set(NNZ)

    mesh = plsc.VectorSubcoreMesh(core_axis_name="c", subcore_axis_name="s")
    cp = pltpu.CompilerParams()
    if "needs_layout_passes" in pltpu.CompilerParams.__dataclass_fields__:
        cp = dataclasses.replace(cp, needs_layout_passes=False)
    run = functools.partial(
        pl.kernel,
        compiler_params=cp,
        out_type=jax.ShapeDtypeStruct((B, OUT_PAD), jnp.float32),
        mesh=mesh,
        scratch_types=[
            pltpu.VMEM((IN_LEN,), jnp.float32),
            pltpu.VMEM((OUT_PAD,), jnp.float32),
            pltpu.VMEM((CHUNK,), jnp.int32),
            pltpu.VMEM((CHUNK,), jnp.int32),
            pltpu.VMEM((CHUNK,), jnp.float32),
            pltpu.VMEM((CHUNK,), jnp.int32),
            pltpu.VMEM((CHUNK,), jnp.int32),
            pltpu.VMEM((CHUNK,), jnp.float32),
            pltpu.VMEM((LANES,), jnp.int32),
            pltpu.VMEM((HALF_LEN,), jnp.float32),
            pltpu.SemaphoreType.DMA,
            pltpu.SemaphoreType.DMA,
        ],
    )(_body)
    outp = run(x2, mask_rows, mask_cols, kernel, bias_pad, off)
    return outp[:, :OUT_LEN].reshape(B, OUT_LEN, 1)


# speedup vs baseline: 12.9266x; 1.8263x over previous
import dataclasses
import functools

import jax
import jax.numpy as jnp
from jax import lax
from jax.experimental import pallas as pl
from jax.experimental.pallas import tpu as pltpu
from jax.experimental.pallas import tpu_sc as plsc

B = 16
IN_LEN = 50000
OUT_LEN = 5000
NNZ = 1600000
NCORES = 2
LANES = 16
HALF_LEN = 2560
OUT_PAD = 2 * HALF_LEN
CHUNK = 8192
STRIDE = CHUNK // LANES
NBLK = -(-NNZ // CHUNK)
NNZ_PAD = NBLK * CHUNK
UNROLL = 4


def _body(x_hbm, rows_hbm, cols_hbm, w_hbm, bias_hbm, off_hbm, out_hbm,
          xb, accv, rbuf0, cbuf0, wbuf0, rbuf1, cbuf1, wbuf1, offv, bbuf,
          sem_a, sem_b):
    c_idx = lax.axis_index("c")
    s_idx = lax.axis_index("s")
    wid = s_idx * NCORES + c_idx
    b = wid % B
    h = wid // B

    pltpu.sync_copy(x_hbm.at[b], xb)
    pltpu.sync_copy(off_hbm, offv)

    iot = lax.iota(jnp.int32, LANES)
    iot_s = iot * STRIDE
    ov = offv[...]
    n_lo = jnp.sum(jnp.where(iot == h, ov, 0))
    n_hi = jnp.sum(jnp.where(iot == h + 1, ov, 0))

    c0 = h * HALF_LEN
    pltpu.sync_copy(bias_hbm.at[pl.ds(c0, HALF_LEN)], bbuf)

    @pl.loop(0, HALF_LEN, step=LANES)
    def _init(j):
        accv[pl.ds(c0 + j, LANES)] = bbuf[pl.ds(j, LANES)]

    lo_al = n_lo & ~(CHUNK - 1)
    nchunks = (n_hi - lo_al + CHUNK - 1) // CHUNK

    def copies(k, rb, cb, wb, sem):
        base = pl.multiple_of(lo_al + k * CHUNK, CHUNK)
        return (
            pltpu.make_async_copy(rows_hbm.at[pl.ds(base, CHUNK)], rb, sem),
            pltpu.make_async_copy(cols_hbm.at[pl.ds(base, CHUNK)], cb, sem),
            pltpu.make_async_copy(w_hbm.at[pl.ds(base, CHUNK)], wb, sem),
        )

    def issue(k, rb, cb, wb, sem):
        for c in copies(k, rb, cb, wb, sem):
            c.start()

    def drain(k, rb, cb, wb, sem):
        for c in copies(k, rb, cb, wb, sem):
            c.wait()

    def compute(k, rb, cb, wb):
        start = lo_al + k * CHUNK
        interior = (start >= n_lo) & (start + CHUNK <= n_hi)

        @pl.when(interior)
        def _fast():
            @pl.loop(0, CHUNK, step=UNROLL * LANES)
            def _grp(j):
                for u in range(UNROLL):
                    sl = pl.ds(j + u * LANES, LANES)
                    xv = plsc.load_gather(xb, [rb[sl]])
                    plsc.addupdate_scatter(accv, [cb[sl]], xv * wb[sl])

        @pl.when(jnp.logical_not(interior))
        def _masked():
            lb = jnp.maximum(n_lo, start)

            @pl.loop(0, CHUNK, step=LANES)
            def _grp(j):
                g = (start + (j >> 4)) + iot_s
                m = (g >= lb) & (g < n_hi)
                sl = pl.ds(j, LANES)
                xv = plsc.load_gather(xb, [rb[sl]])
                plsc.addupdate_scatter(accv, [cb[sl]], xv * wb[sl], mask=m)

    issue(0, rbuf0, cbuf0, wbuf0, sem_a)
    npairs = (nchunks + 1) // 2

    def pair(p, carry):
        k0 = 2 * p
        drain(k0, rbuf0, cbuf0, wbuf0, sem_a)
        issue(k0 + 1, rbuf1, cbuf1, wbuf1, sem_b)
        compute(k0, rbuf0, cbuf0, wbuf0)
        drain(k0 + 1, rbuf1, cbuf1, wbuf1, sem_b)
        issue(k0 + 2, rbuf0, cbuf0, wbuf0, sem_a)
        compute(k0 + 1, rbuf1, cbuf1, wbuf1)
        return carry

    lax.fori_loop(0, npairs, pair, 0)
    drain(2 * npairs, rbuf0, cbuf0, wbuf0, sem_a)

    pltpu.sync_copy(accv.at[pl.ds(c0, HALF_LEN)],
                    out_hbm.at[b, pl.ds(c0, HALF_LEN)])


def _stripe(a):
    a = jnp.pad(a, (0, NNZ_PAD - NNZ))
    return a.reshape(NBLK, LANES, STRIDE).transpose(0, 2, 1).reshape(-1)


@jax.jit
def kernel(x, mask_rows, mask_cols, kernel, bias):
    x2 = x.reshape(B, IN_LEN)
    bias_pad = jnp.pad(bias[:, 0], (0, OUT_PAD - OUT_LEN))
    mid = jnp.searchsorted(mask_cols, HALF_LEN).astype(jnp.int32)
    off = jnp.zeros((LANES,), jnp.int32)
    off = off.at[1].set(mid)
    off = off.at[2:].set(NNZ)

    rows_p = _stripe(mask_rows)
    cols_p = _stripe(mask_cols)
    w_p = _stripe(kernel)

    mesh = plsc.VectorSubcoreMesh(core_axis_name="c", subcore_axis_name="s")
    cp = pltpu.CompilerParams()
    if "needs_layout_passes" in pltpu.CompilerParams.__dataclass_fields__:
        cp = dataclasses.replace(cp, needs_layout_passes=False)
    run = functools.partial(
        pl.kernel,
        compiler_params=cp,
        out_type=jax.ShapeDtypeStruct((B, OUT_PAD), jnp.float32),
        mesh=mesh,
        scratch_types=[
            pltpu.VMEM((IN_LEN,), jnp.float32),
            pltpu.VMEM((OUT_PAD,), jnp.float32),
            pltpu.VMEM((CHUNK,), jnp.int32),
            pltpu.VMEM((CHUNK,), jnp.int32),
            pltpu.VMEM((CHUNK,), jnp.float32),
            pltpu.VMEM((CHUNK,), jnp.int32),
            pltpu.VMEM((CHUNK,), jnp.int32),
            pltpu.VMEM((CHUNK,), jnp.float32),
            pltpu.VMEM((LANES,), jnp.int32),
            pltpu.VMEM((HALF_LEN,), jnp.float32),
            pltpu.SemaphoreType.DMA,
            pltpu.SemaphoreType.DMA,
        ],
    )(_body)
    outp = run(x2, rows_p, cols_p, w_p, bias_pad, off)
    return outp[:, :OUT_LEN].reshape(B, OUT_LEN, 1)


# speedup vs baseline: 23.6740x; 1.8314x over previous
import dataclasses
import functools

import jax
import jax.numpy as jnp
from jax import lax
from jax.experimental import pallas as pl
from jax.experimental.pallas import tpu as pltpu
from jax.experimental.pallas import tpu_sc as plsc

B = 16
IN_LEN = 50000
OUT_LEN = 5000
NNZ = 1600000
NCORES = 2
LANES = 16
HALF_LEN = 2560
OUT_PAD = 2 * HALF_LEN
CHUNK = 8192
STRIDE = CHUNK // LANES
NBLK = -(-NNZ // CHUNK)
NNZ_PAD = NBLK * CHUNK
UNROLL = 4


def _body(x_hbm, rows_hbm, cols_hbm, w_hbm, bias_hbm, off_hbm, out_hbm,
          xb, accv, rbuf0, cbuf0, wbuf0, rbuf1, cbuf1, wbuf1, offv, bbuf,
          sem_a, sem_b):
    c_idx = lax.axis_index("c")
    s_idx = lax.axis_index("s")
    wid = s_idx * NCORES + c_idx
    b = wid % B
    h = wid // B

    pltpu.sync_copy(x_hbm.at[b], xb)
    pltpu.sync_copy(off_hbm, offv)

    iot = lax.iota(jnp.int32, LANES)
    iot_s = iot * STRIDE
    ov = offv[...]
    n_lo = jnp.sum(jnp.where(iot == h, ov, 0))
    n_hi = jnp.sum(jnp.where(iot == h + 1, ov, 0))

    c0 = h * HALF_LEN
    pltpu.sync_copy(bias_hbm.at[pl.ds(c0, HALF_LEN)], bbuf)

    @pl.loop(0, HALF_LEN, step=LANES)
    def _init(j):
        accv[pl.ds(c0 + j, LANES)] = bbuf[pl.ds(j, LANES)]

    lo_al = n_lo & ~(CHUNK - 1)
    nchunks = (n_hi - lo_al + CHUNK - 1) // CHUNK

    def copies(k, rb, cb, wb, sem):
        base = pl.multiple_of(lo_al + k * CHUNK, CHUNK)
        return (
            pltpu.make_async_copy(rows_hbm.at[pl.ds(base, CHUNK)], rb, sem),
            pltpu.make_async_copy(cols_hbm.at[pl.ds(base, CHUNK)], cb, sem),
            pltpu.make_async_copy(w_hbm.at[pl.ds(base, CHUNK)], wb, sem),
        )

    def issue(k, rb, cb, wb, sem):
        for c in copies(k, rb, cb, wb, sem):
            c.start()

    def drain(k, rb, cb, wb, sem):
        for c in copies(k, rb, cb, wb, sem):
            c.wait()

    def compute(k, rb, cb, wb):
        start = lo_al + k * CHUNK
        interior = (start >= n_lo) & (start + CHUNK <= n_hi)

        @pl.when(interior)
        def _fast():
            @plsc.parallel_loop(0, CHUNK, step=LANES, unroll=UNROLL)
            def _grp(j):
                sl = pl.ds(j, LANES)
                xv = plsc.load_gather(xb, [rb[sl]])
                plsc.addupdate_scatter(accv, [cb[sl]], xv * wb[sl])

        @pl.when(jnp.logical_not(interior))
        def _masked():
            lb = jnp.maximum(n_lo, start)

            @pl.loop(0, CHUNK, step=LANES)
            def _grp(j):
                g = (start + (j >> 4)) + iot_s
                m = (g >= lb) & (g < n_hi)
                sl = pl.ds(j, LANES)
                xv = plsc.load_gather(xb, [rb[sl]])
                plsc.addupdate_scatter(accv, [cb[sl]], xv * wb[sl], mask=m)

    issue(0, rbuf0, cbuf0, wbuf0, sem_a)
    npairs = (nchunks + 1) // 2

    def pair(p, carry):
        k0 = 2 * p
        drain(k0, rbuf0, cbuf0, wbuf0, sem_a)
        issue(k0 + 1, rbuf1, cbuf1, wbuf1, sem_b)
        compute(k0, rbuf0, cbuf0, wbuf0)
        drain(k0 + 1, rbuf1, cbuf1, wbuf1, sem_b)
        issue(k0 + 2, rbuf0, cbuf0, wbuf0, sem_a)
        compute(k0 + 1, rbuf1, cbuf1, wbuf1)
        return carry

    lax.fori_loop(0, npairs, pair, 0)
    drain(2 * npairs, rbuf0, cbuf0, wbuf0, sem_a)

    pltpu.sync_copy(accv.at[pl.ds(c0, HALF_LEN)],
                    out_hbm.at[b, pl.ds(c0, HALF_LEN)])


def _stripe(a):
    a = jnp.pad(a, (0, NNZ_PAD - NNZ))
    return a.reshape(NBLK, LANES, STRIDE).transpose(0, 2, 1).reshape(-1)


@jax.jit
def kernel(x, mask_rows, mask_cols, kernel, bias):
    x2 = x.reshape(B, IN_LEN)
    bias_pad = jnp.pad(bias[:, 0], (0, OUT_PAD - OUT_LEN))
    mid = jnp.searchsorted(mask_cols, HALF_LEN).astype(jnp.int32)
    off = jnp.zeros((LANES,), jnp.int32)
    off = off.at[1].set(mid)
    off = off.at[2:].set(NNZ)

    rows_p = _stripe(mask_rows)
    cols_p = _stripe(mask_cols)
    w_p = _stripe(kernel)

    mesh = plsc.VectorSubcoreMesh(core_axis_name="c", subcore_axis_name="s")
    cp = pltpu.CompilerParams()
    if "needs_layout_passes" in pltpu.CompilerParams.__dataclass_fields__:
        cp = dataclasses.replace(cp, needs_layout_passes=False)
    run = functools.partial(
        pl.kernel,
        compiler_params=cp,
        out_type=jax.ShapeDtypeStruct((B, OUT_PAD), jnp.float32),
        mesh=mesh,
        scratch_types=[
            pltpu.VMEM((IN_LEN,), jnp.float32),
            pltpu.VMEM((OUT_PAD,), jnp.float32),
            pltpu.VMEM((CHUNK,), jnp.int32),
            pltpu.VMEM((CHUNK,), jnp.int32),
            pltpu.VMEM((CHUNK,), jnp.float32),
            pltpu.VMEM((CHUNK,), jnp.int32),
            pltpu.VMEM((CHUNK,), jnp.int32),
            pltpu.VMEM((CHUNK,), jnp.float32),
            pltpu.VMEM((LANES,), jnp.int32),
            pltpu.VMEM((HALF_LEN,), jnp.float32),
            pltpu.SemaphoreType.DMA,
            pltpu.SemaphoreType.DMA,
        ],
    )(_body)
    outp = run(x2, rows_p, cols_p, w_p, bias_pad, off)
    return outp[:, :OUT_LEN].reshape(B, OUT_LEN, 1)
